# baseline (speedup 1.0000x reference)
import functools

import jax
import jax.numpy as jnp
from jax import lax
from jax.experimental import pallas as pl
from jax.experimental.pallas import tpu as pltpu
from jax.experimental.pallas import tpu_sc as plsc

B = 16384
F = 26
VOCAB = 100000
VOCABP = 100096
NC = 2
NS = 16
HALF = B // NC
AROWS = HALF // 128


def _sc_body(xt_hbm, tab_hbm, out_hbm, tab_loc, xcol_v, vals_v, idx64_v,
             acc_sh, sem_t, sem_x):
    cid = lax.axis_index("c")
    sid = lax.axis_index("s")

    for k in range(AROWS // 16):
        idx64_v[pl.ds(k * 16, 16)] = lax.iota(jnp.int32, 16) + jnp.broadcast_to(
            jnp.int32(k * 16), (16,)
        )

    @pl.when(sid == NS - 1)
    def _():
        def zrow(r, carry):
            for s2 in range(8):
                vals_v[r, pl.ds(s2 * 16, 16)] = jnp.zeros((16,), jnp.float32)
            return carry

        lax.fori_loop(0, AROWS, zrow, 0)
        pltpu.sync_copy(vals_v, acc_sh)

    def do_field(f, accumulate):
        cp_t = pltpu.async_copy(tab_hbm.at[f], tab_loc, sem_t)
        cp_x = pltpu.async_copy(
            xt_hbm.at[pl.ds(f * B + cid * HALF, HALF)], xcol_v, sem_x
        )
        cp_x.wait()
        cp_t.wait()

        def grow(r, carry):
            for s2 in range(8):
                sl = pl.ds(s2 * 16, 16)
                idx16 = xcol_v[pl.ds(r * 128 + s2 * 16, 16)]
                v16 = plsc.load_gather(tab_loc, [idx16])
                if accumulate:
                    vals_v[r, sl] = vals_v[r, sl] + v16
                else:
                    vals_v[r, sl] = v16
            return carry

        lax.fori_loop(0, AROWS, grow, 0, unroll=2)

    do_field(sid, accumulate=False)

    @pl.when(sid < F - NS)
    def _():
        do_field(sid + NS, accumulate=True)

    plsc.subcore_barrier()
    pltpu.sync_copy(vals_v, acc_sh.at[idx64_v], add=True)
    plsc.subcore_barrier()

    @pl.when(sid == NS - 1)
    def _():
        pltpu.sync_copy(acc_sh, out_hbm.at[cid])


@functools.partial(
    pl.kernel,
    mesh=plsc.VectorSubcoreMesh(core_axis_name="c", subcore_axis_name="s"),
    out_type=jax.ShapeDtypeStruct((NC, AROWS, 128), jnp.float32),
    scratch_types=[
        pltpu.VMEM((VOCAB,), jnp.float32),
        pltpu.VMEM((HALF,), jnp.int32),
        pltpu.VMEM((AROWS, 128), jnp.float32),
        pltpu.VMEM((AROWS,), jnp.int32),
        pltpu.VMEM_SHARED((AROWS, 128), jnp.float32),
        pltpu.SemaphoreType.DMA,
        pltpu.SemaphoreType.DMA,
    ],
    compiler_params=pltpu.CompilerParams(needs_layout_passes=False, use_tc_tiling_on_sc=True),
)
def _sc_call(xt_hbm, tab_hbm, out_hbm, tab_loc, xcol_v, vals_v, idx64_v,
             acc_sh, sem_t, sem_x):
    _sc_body(xt_hbm, tab_hbm, out_hbm, tab_loc, xcol_v, vals_v, idx64_v,
             acc_sh, sem_t, sem_x)


@jax.jit
def kernel(x, tables):
    xt = x.astype(jnp.int32).T.reshape(-1)
    out = _sc_call(xt, tables.reshape(F, VOCAB))
    return out.reshape(B, 1)


# speedup vs baseline: 1.0687x; 1.0687x over previous
import functools

import jax
import jax.numpy as jnp
from jax import lax
from jax.experimental import pallas as pl
from jax.experimental.pallas import tpu as pltpu
from jax.experimental.pallas import tpu_sc as plsc

B = 16384
F = 26
VOCAB = 100000
VOCABP = 100096
NC = 2
NS = 16
HALF = B // NC
AROWS = HALF // 128


def _sc_body(xt_hbm, tab_hbm, out_hbm, tab_loc, xcol_v, vals_v, idx64_v,
             acc_sh, sem_t, sem_x):
    cid = lax.axis_index("c")
    sid = lax.axis_index("s")

    for k in range(AROWS // 16):
        idx64_v[pl.ds(k * 16, 16)] = lax.iota(jnp.int32, 16) + jnp.broadcast_to(
            jnp.int32(k * 16), (16,)
        )

    @pl.when(sid == 0)
    def _():
        def zrow(r, carry):
            for s2 in range(8):
                vals_v[r, pl.ds(s2 * 16, 16)] = jnp.zeros((16,), jnp.float32)
            return carry

        lax.fori_loop(0, AROWS, zrow, 0)
        pltpu.sync_copy(vals_v, acc_sh)

    plsc.subcore_barrier()

    def do_field(f, accumulate):
        cp_t = pltpu.async_copy(tab_hbm.at[f], tab_loc, sem_t)
        cp_x = pltpu.async_copy(
            xt_hbm.at[pl.ds(f * B + cid * HALF, HALF)], xcol_v, sem_x
        )
        cp_x.wait()
        cp_t.wait()

        def grow(r, carry):
            for s2 in range(8):
                sl = pl.ds(s2 * 16, 16)
                idx16 = xcol_v[pl.ds(r * 128 + s2 * 16, 16)]
                v16 = plsc.load_gather(tab_loc, [idx16])
                if accumulate:
                    vals_v[r, sl] = vals_v[r, sl] + v16
                else:
                    vals_v[r, sl] = v16
            return carry

        lax.fori_loop(0, AROWS, grow, 0)

    do_field(sid, accumulate=False)

    @pl.when(sid < F - NS)
    def _():
        do_field(sid + NS, accumulate=True)

    pltpu.sync_copy(vals_v, acc_sh.at[idx64_v], add=True)
    plsc.subcore_barrier()

    @pl.when(sid == NS - 1)
    def _():
        pltpu.sync_copy(acc_sh, out_hbm.at[cid])


@functools.partial(
    pl.kernel,
    mesh=plsc.VectorSubcoreMesh(core_axis_name="c", subcore_axis_name="s"),
    out_type=jax.ShapeDtypeStruct((NC, AROWS, 128), jnp.float32),
    scratch_types=[
        pltpu.VMEM((VOCAB,), jnp.float32),
        pltpu.VMEM((HALF,), jnp.int32),
        pltpu.VMEM((AROWS, 128), jnp.float32),
        pltpu.VMEM((AROWS,), jnp.int32),
        pltpu.VMEM_SHARED((AROWS, 128), jnp.float32),
        pltpu.SemaphoreType.DMA,
        pltpu.SemaphoreType.DMA,
    ],
    compiler_params=pltpu.CompilerParams(needs_layout_passes=False, use_tc_tiling_on_sc=True),
)
def _sc_call(xt_hbm, tab_hbm, out_hbm, tab_loc, xcol_v, vals_v, idx64_v,
             acc_sh, sem_t, sem_x):
    _sc_body(xt_hbm, tab_hbm, out_hbm, tab_loc, xcol_v, vals_v, idx64_v,
             acc_sh, sem_t, sem_x)


@jax.jit
def kernel(x, tables):
    xt = x.astype(jnp.int32).T.reshape(-1)
    out = _sc_call(xt, tables.reshape(F, VOCAB))
    return out.reshape(B, 1)


# speedup vs baseline: 1.0728x; 1.0038x over previous
import functools

import jax
import jax.numpy as jnp
from jax import lax
from jax.experimental import pallas as pl
from jax.experimental.pallas import tpu as pltpu
from jax.experimental.pallas import tpu_sc as plsc

B = 16384
F = 26
VOCAB = 100000
VOCABP = 100096
NC = 2
NS = 16
HALF = B // NC
AROWS = HALF // 128


def _sc_body(xt_hbm, tab_hbm, out_hbm, tab_loc, xcol_v, vals_v, idx64_v,
             acc_sh, sem_t, sem_x):
    cid = lax.axis_index("c")
    sid = lax.axis_index("s")

    for k in range(AROWS // 16):
        idx64_v[pl.ds(k * 16, 16)] = lax.iota(jnp.int32, 16) + jnp.broadcast_to(
            jnp.int32(k * 16), (16,)
        )

    @pl.when(sid == 0)
    def _():
        def zrow(r, carry):
            for s2 in range(8):
                vals_v[r, pl.ds(s2 * 16, 16)] = jnp.zeros((16,), jnp.float32)
            return carry

        lax.fori_loop(0, AROWS, zrow, 0)
        pltpu.sync_copy(vals_v, acc_sh)

    plsc.subcore_barrier()

    def do_field(f, accumulate):
        cp_t = pltpu.async_copy(tab_hbm.at[f], tab_loc, sem_t)
        cp_x = pltpu.async_copy(
            xt_hbm.at[pl.ds(f * B + cid * HALF, HALF)], xcol_v, sem_x
        )
        cp_x.wait()
        cp_t.wait()

        def grow(r, carry):
            for s2 in range(8):
                sl = pl.ds(s2 * 16, 16)
                idx16 = xcol_v[pl.ds(r * 128 + s2 * 16, 16)]
                v16 = plsc.load_gather(tab_loc, [idx16])
                if accumulate:
                    vals_v[r, sl] = vals_v[r, sl] + v16
                else:
                    vals_v[r, sl] = v16
            return carry

        lax.fori_loop(0, AROWS, grow, 0)

    do_field(sid, accumulate=False)

    @pl.when(sid < F - NS)
    def _():
        do_field(sid + NS, accumulate=True)

    pltpu.sync_copy(vals_v, acc_sh.at[idx64_v], add=True)
    plsc.subcore_barrier()

    @pl.when(sid == 0)
    def _():
        pltpu.sync_copy(acc_sh, out_hbm.at[cid])


@functools.partial(
    pl.kernel,
    mesh=plsc.VectorSubcoreMesh(core_axis_name="c", subcore_axis_name="s"),
    out_type=jax.ShapeDtypeStruct((NC, AROWS, 128), jnp.float32),
    scratch_types=[
        pltpu.VMEM((VOCAB,), jnp.float32),
        pltpu.VMEM((HALF,), jnp.int32),
        pltpu.VMEM((AROWS, 128), jnp.float32),
        pltpu.VMEM((AROWS,), jnp.int32),
        pltpu.VMEM_SHARED((AROWS, 128), jnp.float32),
        pltpu.SemaphoreType.DMA,
        pltpu.SemaphoreType.DMA,
    ],
    compiler_params=pltpu.CompilerParams(needs_layout_passes=False, use_tc_tiling_on_sc=True),
)
def _sc_call(xt_hbm, tab_hbm, out_hbm, tab_loc, xcol_v, vals_v, idx64_v,
             acc_sh, sem_t, sem_x):
    _sc_body(xt_hbm, tab_hbm, out_hbm, tab_loc, xcol_v, vals_v, idx64_v,
             acc_sh, sem_t, sem_x)


@jax.jit
def kernel(x, tables):
    xt = x.astype(jnp.int32).T.reshape(-1)
    out = _sc_call(xt, tables.reshape(F, VOCAB))
    return out.reshape(B, 1)
